# baseline (speedup 1.0000x reference)
import functools

import jax
import jax.numpy as jnp
from jax import lax
from jax.experimental import pallas as pl
from jax.experimental.pallas import tpu as pltpu
from jax.experimental.pallas import tpu_sc as plsc

B = 64
Q = 16
KV = 32768
ROWS = B * Q

R_SC = 320
R_TC = ROWS - R_SC

NUM_WORKERS = 32
RPW = R_SC // NUM_WORKERS
SLOT = ((RPW + 15) // 16) * 16
LANES = 16
VECS = KV // LANES
BLK = 16
NBLK = VECS // BLK
BIG = 2 ** 20
INT_MAX = 2 ** 31 - 1

TC_BLK_R = 64


def _bfly_max(v, vs_v, lane_iota):
    for off in (1, 2, 4, 8):
        vs_v[...] = v
        ov = plsc.load_gather(vs_v, [lane_iota ^ off])
        v = jnp.maximum(v, ov)
    return v


def _bfly_min_i32(e, es_v, lane_iota):
    for off in (1, 2, 4, 8):
        es_v[...] = e
        oe = plsc.load_gather(es_v, [lane_iota ^ off])
        e = jnp.minimum(e, oe)
    return e


def _row_argmax(buf, vs_v, es_v, lane_iota):
    neg_inf = jnp.full((LANES,), -jnp.inf, dtype=jnp.float32)
    zeros = jnp.zeros((LANES,), jnp.int32)

    def blk_step(b, carry):
        gmax, gblk = carry
        off = b * (BLK * LANES)
        vs = [buf[pl.ds(off + k * LANES, LANES)] for k in range(BLK)]
        while len(vs) > 1:
            vs = [jnp.maximum(vs[2 * i], vs[2 * i + 1])
                  for i in range(len(vs) // 2)]
        m = vs[0]
        pred = m > gmax
        gmax = jnp.where(pred, m, gmax)
        gblk = jnp.where(pred, jnp.full((LANES,), b, jnp.int32), gblk)
        return gmax, gblk

    gmax, gblk = lax.fori_loop(0, NBLK, blk_step, (neg_inf, zeros), unroll=4)

    mvec = _bfly_max(gmax, vs_v, lane_iota)
    cand = jnp.where(gmax == mvec, gblk, BIG)
    bsel = _bfly_min_i32(cand, es_v, lane_iota)
    bsel_s = bsel[0]

    off = bsel_s * (BLK * LANES)
    bp = jnp.full((LANES,), BIG, jnp.int32)
    for k in range(BLK):
        v = buf[pl.ds(off + k * LANES, LANES)]
        hit = v == mvec
        bp = jnp.minimum(bp, jnp.where(hit, jnp.int32(k), BIG))

    e = ((bsel * BLK + bp) * LANES) + lane_iota
    e = jnp.where(bp < BIG, e, INT_MAX)
    return _bfly_min_i32(e, es_v, lane_iota)


def _argmax_body(x_hbm, out_hbm, buf0, buf1, res_v, vs_v, es_v, sem0, sem1):
    c = lax.axis_index("c")
    s = lax.axis_index("s")
    wid = s * 2 + c
    base = wid * RPW

    lane_iota = lax.iota(jnp.int32, LANES)

    def compute(buf, r_local):
        e = _row_argmax(buf, vs_v, es_v, lane_iota)
        goff = (r_local // LANES) * LANES
        cur = res_v[pl.ds(goff, LANES)]
        res_v[pl.ds(goff, LANES)] = jnp.where(
            lane_iota == (r_local - goff), e, cur)

    pltpu.make_async_copy(x_hbm.at[base], buf0, sem0).start()

    def pair(p, _):
        r0 = 2 * p
        pltpu.make_async_copy(x_hbm.at[base + r0 + 1], buf1, sem1).start()
        pltpu.make_async_copy(x_hbm.at[base + r0], buf0, sem0).wait()
        compute(buf0, r0)

        @pl.when(p < RPW // 2 - 1)
        def _():
            pltpu.make_async_copy(x_hbm.at[base + r0 + 2], buf0, sem0).start()

        pltpu.make_async_copy(x_hbm.at[base + r0 + 1], buf1, sem1).wait()
        compute(buf1, r0 + 1)
        return 0

    lax.fori_loop(0, RPW // 2, pair, 0)

    pltpu.sync_copy(res_v, out_hbm.at[pl.ds(wid * SLOT, SLOT)])


def _sc_argmax(x):
    mesh = plsc.VectorSubcoreMesh(core_axis_name="c", subcore_axis_name="s")
    kern = functools.partial(
        pl.kernel,
        mesh=mesh,
        compiler_params=pltpu.CompilerParams(needs_layout_passes=False),
        out_type=jax.ShapeDtypeStruct((NUM_WORKERS * SLOT,), jnp.int32),
        scratch_types=[
            pltpu.VMEM((KV,), jnp.float32),
            pltpu.VMEM((KV,), jnp.float32),
            pltpu.VMEM((SLOT,), jnp.int32),
            pltpu.VMEM((LANES,), jnp.float32),
            pltpu.VMEM((LANES,), jnp.int32),
            pltpu.SemaphoreType.DMA,
            pltpu.SemaphoreType.DMA,
        ],
    )(_argmax_body)
    return kern(x)


def _tc_body(x_ref, o_ref):
    x = x_ref[...]
    m = jnp.max(x, axis=1, keepdims=True)
    idx = lax.broadcasted_iota(jnp.int32, x.shape, 1)
    cand = jnp.where(x == m, idx, INT_MAX)
    o_ref[...] = jnp.min(cand, axis=1, keepdims=True)


def _tc_argmax(x):
    row0 = R_SC // TC_BLK_R
    return pl.pallas_call(
        _tc_body,
        grid=(R_TC // TC_BLK_R,),
        in_specs=[pl.BlockSpec((TC_BLK_R, KV), lambda i: (i + row0, 0))],
        out_specs=pl.BlockSpec((TC_BLK_R, 1), lambda i: (i, 0)),
        out_shape=jax.ShapeDtypeStruct((R_TC, 1), jnp.int32),
    )(x)


@jax.jit
def _argmax_rows(x):
    out_sc = _sc_argmax(x).reshape(NUM_WORKERS, SLOT)[:, :RPW].reshape(-1)
    out_tc = _tc_argmax(x)
    return jnp.concatenate([out_sc, out_tc[:, 0]])


def kernel(xyz):
    x = xyz.reshape(ROWS, KV)
    out = _argmax_rows(x)
    return out.reshape(B, Q)


# speedup vs baseline: 1.0125x; 1.0125x over previous
import functools

import jax
import jax.numpy as jnp
from jax import lax
from jax.experimental import pallas as pl
from jax.experimental.pallas import tpu as pltpu
from jax.experimental.pallas import tpu_sc as plsc

B = 64
Q = 16
KV = 32768
ROWS = B * Q

R_SC = 448
R_TC = ROWS - R_SC

NUM_WORKERS = 32
RPW = R_SC // NUM_WORKERS
SLOT = ((RPW + 15) // 16) * 16
LANES = 16
VECS = KV // LANES
BLK = 16
NBLK = VECS // BLK
BIG = 2 ** 20
INT_MAX = 2 ** 31 - 1

TC_BLK_R = 64


def _bfly_max(v, vs_v, lane_iota):
    for off in (1, 2, 4, 8):
        vs_v[...] = v
        ov = plsc.load_gather(vs_v, [lane_iota ^ off])
        v = jnp.maximum(v, ov)
    return v


def _bfly_min_i32(e, es_v, lane_iota):
    for off in (1, 2, 4, 8):
        es_v[...] = e
        oe = plsc.load_gather(es_v, [lane_iota ^ off])
        e = jnp.minimum(e, oe)
    return e


def _row_argmax(buf, vs_v, es_v, lane_iota):
    neg_inf = jnp.full((LANES,), -jnp.inf, dtype=jnp.float32)
    zeros = jnp.zeros((LANES,), jnp.int32)

    def blk_step(b, carry):
        gmax, gblk = carry
        off = b * (BLK * LANES)
        vs = [buf[pl.ds(off + k * LANES, LANES)] for k in range(BLK)]
        while len(vs) > 1:
            vs = [jnp.maximum(vs[2 * i], vs[2 * i + 1])
                  for i in range(len(vs) // 2)]
        m = vs[0]
        pred = m > gmax
        gmax = jnp.where(pred, m, gmax)
        gblk = jnp.where(pred, jnp.full((LANES,), b, jnp.int32), gblk)
        return gmax, gblk

    gmax, gblk = lax.fori_loop(0, NBLK, blk_step, (neg_inf, zeros), unroll=4)

    mvec = _bfly_max(gmax, vs_v, lane_iota)
    cand = jnp.where(gmax == mvec, gblk, BIG)
    bsel = _bfly_min_i32(cand, es_v, lane_iota)
    bsel_s = bsel[0]

    off = bsel_s * (BLK * LANES)
    bp = jnp.full((LANES,), BIG, jnp.int32)
    for k in range(BLK):
        v = buf[pl.ds(off + k * LANES, LANES)]
        hit = v == mvec
        bp = jnp.minimum(bp, jnp.where(hit, jnp.int32(k), BIG))

    e = ((bsel * BLK + bp) * LANES) + lane_iota
    e = jnp.where(bp < BIG, e, INT_MAX)
    return _bfly_min_i32(e, es_v, lane_iota)


def _argmax_body(x_hbm, out_hbm, buf0, buf1, res_v, vs_v, es_v, sem0, sem1):
    c = lax.axis_index("c")
    s = lax.axis_index("s")
    wid = s * 2 + c
    base = wid * RPW

    lane_iota = lax.iota(jnp.int32, LANES)

    def compute(buf, r_local):
        e = _row_argmax(buf, vs_v, es_v, lane_iota)
        goff = (r_local // LANES) * LANES
        cur = res_v[pl.ds(goff, LANES)]
        res_v[pl.ds(goff, LANES)] = jnp.where(
            lane_iota == (r_local - goff), e, cur)

    pltpu.make_async_copy(x_hbm.at[base], buf0, sem0).start()

    def pair(p, _):
        r0 = 2 * p
        pltpu.make_async_copy(x_hbm.at[base + r0 + 1], buf1, sem1).start()
        pltpu.make_async_copy(x_hbm.at[base + r0], buf0, sem0).wait()
        compute(buf0, r0)

        @pl.when(p < RPW // 2 - 1)
        def _():
            pltpu.make_async_copy(x_hbm.at[base + r0 + 2], buf0, sem0).start()

        pltpu.make_async_copy(x_hbm.at[base + r0 + 1], buf1, sem1).wait()
        compute(buf1, r0 + 1)
        return 0

    lax.fori_loop(0, RPW // 2, pair, 0)

    pltpu.sync_copy(res_v, out_hbm.at[pl.ds(wid * SLOT, SLOT)])


def _sc_argmax(x):
    mesh = plsc.VectorSubcoreMesh(core_axis_name="c", subcore_axis_name="s")
    kern = functools.partial(
        pl.kernel,
        mesh=mesh,
        compiler_params=pltpu.CompilerParams(needs_layout_passes=False),
        out_type=jax.ShapeDtypeStruct((NUM_WORKERS * SLOT,), jnp.int32),
        scratch_types=[
            pltpu.VMEM((KV,), jnp.float32),
            pltpu.VMEM((KV,), jnp.float32),
            pltpu.VMEM((SLOT,), jnp.int32),
            pltpu.VMEM((LANES,), jnp.float32),
            pltpu.VMEM((LANES,), jnp.int32),
            pltpu.SemaphoreType.DMA,
            pltpu.SemaphoreType.DMA,
        ],
    )(_argmax_body)
    return kern(x)


def _tc_body(x_ref, o_ref):
    x = x_ref[...]
    m = jnp.max(x, axis=1, keepdims=True)
    idx = lax.broadcasted_iota(jnp.int32, x.shape, 1)
    cand = jnp.where(x == m, idx, INT_MAX)
    o_ref[...] = jnp.min(cand, axis=1, keepdims=True)


def _tc_argmax(x):
    row0 = R_SC // TC_BLK_R
    return pl.pallas_call(
        _tc_body,
        grid=(R_TC // TC_BLK_R,),
        in_specs=[pl.BlockSpec((TC_BLK_R, KV), lambda i: (i + row0, 0))],
        out_specs=pl.BlockSpec((TC_BLK_R, 1), lambda i: (i, 0)),
        out_shape=jax.ShapeDtypeStruct((R_TC, 1), jnp.int32),
    )(x)


@jax.jit
def _argmax_rows(x):
    out_sc = _sc_argmax(x).reshape(NUM_WORKERS, SLOT)[:, :RPW].reshape(-1)
    out_tc = _tc_argmax(x)
    return jnp.concatenate([out_sc, out_tc[:, 0]])


def kernel(xyz):
    x = xyz.reshape(ROWS, KV)
    out = _argmax_rows(x)
    return out.reshape(B, Q)
